# Initial kernel scaffold; baseline (speedup 1.0000x reference)
import jax
import jax.numpy as jnp
from jax.experimental import pallas as pl


def kernel(emb0, emb1, emb2, emb3, lap0, lap1, lap2, lap3, bnd1, bnd2, bnd3, order, idx, rel, params):
    raise NotImplementedError("write your pallas kernel here")


# speedup vs baseline: 1.1894x; 1.1894x over previous
import functools

import jax
import jax.numpy as jnp
from jax import lax
from jax.experimental import pallas as pl
from jax.experimental.pallas import tpu as pltpu
from jax.experimental.pallas import tpu_sc as plsc

_F32 = jnp.float32


def _sc_gather_rows(table, idx):
    V, D = table.shape
    B = idx.shape[0]
    info = plsc.get_sparse_core_info()
    NC, NS = info.num_cores, info.num_subcores
    NW = NC * NS
    b_per_w = B // NW
    mesh = plsc.VectorSubcoreMesh(core_axis_name="c", subcore_axis_name="s")

    @functools.partial(
        pl.kernel, mesh=mesh,
        out_type=jax.ShapeDtypeStruct((B, D), _F32),
        scratch_types=[
            pltpu.VMEM((b_per_w,), jnp.int32),
            pltpu.VMEM((b_per_w, D), _F32),
            pltpu.SemaphoreType.DMA,
        ],
    )
    def k(table_hbm, idx_hbm, out_hbm, idx_v, rows_v, sem):
        wid = lax.axis_index("s") * NC + lax.axis_index("c")
        base = wid * b_per_w
        pltpu.sync_copy(idx_hbm.at[pl.ds(base, b_per_w)], idx_v)
        pltpu.async_copy(table_hbm.at[idx_v], rows_v, sem).wait()
        pltpu.sync_copy(rows_v, out_hbm.at[pl.ds(base, b_per_w)])

    return k(table, idx)


def _proj(x, ws, block_rows=256):
    N, di = x.shape
    K = len(ws)

    def body(*refs):
        x_ref = refs[0]
        xb = x_ref[...]
        for w_ref, o_ref in zip(refs[1:1 + K], refs[1 + K:]):
            o_ref[...] = jnp.dot(xb, w_ref[...], preferred_element_type=_F32)

    in_specs = [pl.BlockSpec((block_rows, di), lambda i: (i, 0))]
    in_specs += [pl.BlockSpec(w.shape, lambda i: (0, 0)) for w in ws]
    out_specs = [pl.BlockSpec((block_rows, w.shape[1]), lambda i: (i, 0))
                 for w in ws]
    out_shape = [jax.ShapeDtypeStruct((N, w.shape[1]), _F32) for w in ws]
    outs = pl.pallas_call(
        body,
        grid=(N // block_rows,),
        in_specs=in_specs,
        out_specs=out_specs,
        out_shape=out_shape,
    )(x, *ws)
    return list(outs)


def _attn(L, h, a1, a2, bd, pd, bu, pu, block_rows=256):
    N, do = h.shape
    has_d = bd is not None
    has_u = bu is not None

    def body(*refs):
        it = iter(refs)
        L_ref, h_ref, a1_ref, a2_ref = next(it), next(it), next(it), next(it)
        bd_ref = next(it) if has_d else None
        pd_ref = next(it) if has_d else None
        bu_ref = next(it) if has_u else None
        pu_ref = next(it) if has_u else None
        o_ref = next(it)

        i = pl.program_id(0)
        hf = h_ref[...]
        hb = h_ref[pl.ds(i * block_rows, block_rows), :]
        s1 = lax.dot_general(hb, a1_ref[...], (((1,), (1,)), ((), ())),
                             preferred_element_type=_F32)
        s2 = lax.dot_general(a2_ref[...], hf, (((1,), (1,)), ((), ())),
                             preferred_element_type=_F32)
        e = s1 + s2
        e = jnp.where(e >= 0, e, 0.2 * e)
        e = jnp.where(L_ref[...] != 0, e, -1e9)
        m = jnp.max(e, axis=1, keepdims=True)
        w = jnp.exp(e - m)
        den = jnp.sum(w, axis=1, keepdims=True)
        acc = jnp.dot(w, hf, preferred_element_type=_F32) / den
        if has_d:
            acc += lax.dot_general(bd_ref[...], pd_ref[...],
                                   (((0,), (0,)), ((), ())),
                                   preferred_element_type=_F32)
        if has_u:
            acc += jnp.dot(bu_ref[...], pu_ref[...],
                           preferred_element_type=_F32)
        o_ref[...] = jnp.maximum(acc, 0.0)

    in_specs = [
        pl.BlockSpec((block_rows, N), lambda i: (i, 0)),
        pl.BlockSpec((N, do), lambda i: (0, 0)),
        pl.BlockSpec((1, do), lambda i: (0, 0)),
        pl.BlockSpec((1, do), lambda i: (0, 0)),
    ]
    args = [L, h, a1, a2]
    if has_d:
        np_ = bd.shape[0]
        in_specs += [pl.BlockSpec((np_, block_rows), lambda i: (0, i)),
                     pl.BlockSpec((np_, do), lambda i: (0, 0))]
        args += [bd, pd]
    if has_u:
        nn_ = bu.shape[1]
        in_specs += [pl.BlockSpec((block_rows, nn_), lambda i: (i, 0)),
                     pl.BlockSpec((nn_, do), lambda i: (0, 0))]
        args += [bu, pu]

    return pl.pallas_call(
        body,
        grid=(N // block_rows,),
        in_specs=in_specs,
        out_specs=pl.BlockSpec((block_rows, do), lambda i: (i, 0)),
        out_shape=jax.ShapeDtypeStruct((N, do), _F32),
    )(*args)


def _final(Lg, hg, h0, a1, a2, Bg, pu, wrel, brel):
    B = Lg.shape[0]
    N, do = h0.shape
    C = wrel.shape[1]

    def body(Lg_ref, hg_ref, h0_ref, a1_ref, a2_ref, Bg_ref, pu_ref,
             wrel_ref, brel_ref, o_ref):
        hf = h0_ref[...]
        s1 = lax.dot_general(hg_ref[...], a1_ref[...], (((1,), (1,)), ((), ())),
                             preferred_element_type=_F32)
        s2 = lax.dot_general(a2_ref[...], hf, (((1,), (1,)), ((), ())),
                             preferred_element_type=_F32)
        e = s1 + s2
        e = jnp.where(e >= 0, e, 0.2 * e)
        e = jnp.where(Lg_ref[...] != 0, e, -1e9)
        m = jnp.max(e, axis=1, keepdims=True)
        w = jnp.exp(e - m)
        den = jnp.sum(w, axis=1, keepdims=True)
        acc = jnp.dot(w, hf, preferred_element_type=_F32) / den
        acc += jnp.dot(Bg_ref[...], pu_ref[...], preferred_element_type=_F32)
        acc = jnp.maximum(acc, 0.0)
        o_ref[...] = (jnp.dot(acc, wrel_ref[...], preferred_element_type=_F32)
                      + brel_ref[...])

    return pl.pallas_call(
        body,
        out_shape=jax.ShapeDtypeStruct((B, C), _F32),
    )(Lg, hg, h0, a1, a2, Bg, pu, wrel, brel)


def _split_a(lp):
    a = lp["a"]
    do = a.shape[0] // 2
    return a[:do].reshape(1, do), a[do:].reshape(1, do)


def kernel(emb0, emb1, emb2, emb3, lap0, lap1, lap2, lap3,
           bnd1, bnd2, bnd3, order, idx, rel, params):
    del order
    idx = idx.astype(jnp.int32)
    laps = [lap0, lap1, lap2, lap3]
    bnds = [None, bnd1, bnd2, bnd3]

    Lg = _sc_gather_rows(lap0, idx)
    Bg = _sc_gather_rows(bnd1, idx)

    xs = [emb0, emb1, emb2, emb3]

    need = {
        1: {0: "h d", 1: "h d u", 2: "h d u", 3: "h u"},
        2: {0: "h d", 1: "h d u", 2: "h u", 3: "u"},
        3: {0: "h d", 1: "h u", 2: "u"},
    }
    orders_per_layer = {1: (0, 1, 2, 3), 2: (0, 1, 2), 3: (0, 1)}

    for lnum in (1, 2, 3):
        lp = params["l%d" % lnum]
        a1, a2 = _split_a(lp)
        h, hd, hu = {}, {}, {}
        for i, spec in need[lnum].items():
            toks = spec.split()
            ws, dsts = [], []
            if "h" in toks:
                ws.append(lp["W"]); dsts.append((h, i))
            if "d" in toks:
                ws.append(lp["Wd"]); dsts.append((hd, i))
            if "u" in toks:
                ws.append(lp["Wu"]); dsts.append((hu, i))
            outs = _proj(xs[i], ws)
            for (dct, key), o in zip(dsts, outs):
                dct[key] = o
        nxt = [None, None, None, None]
        for i in orders_per_layer[lnum]:
            bd = bnds[i] if i > 0 else None
            pd = hd.get(i - 1) if i > 0 else None
            bu = bnds[i + 1] if i < 3 else None
            pu = hu.get(i + 1) if i < 3 else None
            nxt[i] = _attn(laps[i], h[i], a1, a2, bd, pd, bu, pu)
        xs = nxt

    lp = params["l4"]
    a1, a2 = _split_a(lp)
    (h0,) = _proj(xs[0], [lp["W"]])
    (pu1,) = _proj(xs[1], [lp["Wu"]])
    hg = _sc_gather_rows(h0, idx)

    rows = _final(Lg, hg, h0, a1, a2, Bg, pu1,
                  params["W_rel"], params["b_rel"].reshape(1, -1))

    nz = jnp.stack(jnp.nonzero(rel, size=rel.shape[0]), axis=1)
    return rows[nz]
